# baseline (speedup 1.0000x reference)
import functools

import jax
import jax.numpy as jnp
from jax import lax
from jax.experimental import pallas as pl
from jax.experimental.pallas import tpu as pltpu
from jax.experimental.pallas import tpu_sc as plsc

N_NODES = 10000
D = 128
E = 320000

NC = 2
NS = 16
NW = NC * NS
EPW = E // NW
C = 200
CI = 208
NCH = EPW // C
NP = NCH // 2
G = 16


def _sc_kernel(z_hbm, src_hbm, dst_hbm, out_hbm,
               sidx0, didx0, sidx1, didx1,
               srows0, drows0, srows1, drows1,
               outpair, sem0, sem1):
    wid = lax.axis_index("s") * NC + lax.axis_index("c")
    base = wid * EPW

    def start(j, ib_s, ib_d, sb, db, sem):
        pltpu.sync_copy(src_hbm.at[pl.ds(base + j * C, CI)], ib_s)
        pltpu.sync_copy(dst_hbm.at[pl.ds(base + j * C, CI)], ib_d)
        pltpu.async_copy(z_hbm.at[ib_s], sb, sem)
        pltpu.async_copy(z_hbm.at[ib_d], db, sem)

    def wait_rows(ib_s, ib_d, sb, db, sem):
        pltpu.make_async_copy(z_hbm.at[ib_s], sb, sem).wait()
        pltpu.make_async_copy(z_hbm.at[ib_d], db, sem).wait()

    iota = lax.iota(jnp.int32, G)

    def compute(q, sb, db):
        @plsc.parallel_loop(0, CI // G, 1, unroll=1)
        def group_body(t):
            rows = t * G + iota
            accs = [jnp.zeros((G,), jnp.float32) for _ in range(4)]
            for k in range(D):
                col = jnp.full((G,), k, jnp.int32)
                prod = (plsc.load_gather(sb, [rows, col])
                        * plsc.load_gather(db, [rows, col]))
                accs[k % 4] = accs[k % 4] + prod
            tot = (accs[0] + accs[1]) + (accs[2] + accs[3])
            outpair[pl.ds(q * C + t * G, G)] = 1.0 / (1.0 + jnp.exp(-tot))

    start(0, sidx0, didx0, srows0, drows0, sem0)

    def pair_body(p, _):
        j0 = 2 * p
        j1 = j0 + 1
        start(j1, sidx1, didx1, srows1, drows1, sem1)

        wait_rows(sidx0, didx0, srows0, drows0, sem0)
        compute(0, srows0, drows0)

        jn = lax.rem(j0 + 2, NCH)
        start(jn, sidx0, didx0, srows0, drows0, sem0)

        wait_rows(sidx1, didx1, srows1, drows1, sem1)
        compute(1, srows1, drows1)

        pltpu.sync_copy(outpair.at[pl.ds(0, 2 * C)],
                        out_hbm.at[pl.ds(base + p * 2 * C, 2 * C)])
        return 0

    lax.fori_loop(0, NP, pair_body, 0)
    wait_rows(sidx0, didx0, srows0, drows0, sem0)


@jax.jit
def kernel(z, edge_index):
    edge_index = edge_index.astype(jnp.int32)
    pad = jnp.zeros((2, CI - C), jnp.int32)
    edge_index = jnp.concatenate([edge_index, pad], axis=1)
    src = edge_index[0]
    dst = edge_index[1]

    mesh = plsc.VectorSubcoreMesh(core_axis_name="c", subcore_axis_name="s")
    run = functools.partial(
        pl.kernel,
        out_type=jax.ShapeDtypeStruct((E,), jnp.float32),
        mesh=mesh,
        compiler_params=pltpu.CompilerParams(needs_layout_passes=False),
        scratch_types=[
            pltpu.VMEM((CI,), jnp.int32),
            pltpu.VMEM((CI,), jnp.int32),
            pltpu.VMEM((CI,), jnp.int32),
            pltpu.VMEM((CI,), jnp.int32),
            pltpu.VMEM((CI, D), jnp.float32),
            pltpu.VMEM((CI, D), jnp.float32),
            pltpu.VMEM((CI, D), jnp.float32),
            pltpu.VMEM((CI, D), jnp.float32),
            pltpu.VMEM((2 * C + G,), jnp.float32),
            pltpu.SemaphoreType.DMA,
            pltpu.SemaphoreType.DMA,
        ],
    )(_sc_kernel)
    return run(z, src, dst)


# speedup vs baseline: 7.5996x; 7.5996x over previous
import functools

import jax
import jax.numpy as jnp
from jax import lax
from jax.experimental import pallas as pl
from jax.experimental.pallas import tpu as pltpu
from jax.experimental.pallas import tpu_sc as plsc

N_NODES = 10000
D = 128
E = 320000

NC = 2
NS = 16
NW = NC * NS
EPW = E // NW
C = 200
CI = 208
NCH = EPW // C
NP = NCH // 2
G = 16
FR = 17


def _sc_kernel(z_hbm, src_hbm, dst_hbm, out_hbm,
               sidx0, didx0, sidx1, didx1,
               srows0, drows0, srows1, drows1,
               outpair, fold0, fold1, sem0, sem1):
    wid = lax.axis_index("s") * NC + lax.axis_index("c")
    base = wid * EPW

    def start(j, ib_s, ib_d, sb, db, sem):
        pltpu.sync_copy(src_hbm.at[pl.ds(base + j * C, CI)], ib_s)
        pltpu.sync_copy(dst_hbm.at[pl.ds(base + j * C, CI)], ib_d)
        pltpu.async_copy(z_hbm.at[ib_s], sb, sem)
        pltpu.async_copy(z_hbm.at[ib_d], db, sem)

    def wait_rows(ib_s, ib_d, sb, db, sem):
        pltpu.make_async_copy(z_hbm.at[ib_s], sb, sem).wait()
        pltpu.make_async_copy(z_hbm.at[ib_d], db, sem).wait()

    iota = lax.iota(jnp.int32, G)

    def compute(q, sb, db, fold):
        @plsc.parallel_loop(0, CI // G, 1, unroll=1)
        def group_body(t):
            off = t * (G * FR)
            for e in range(G):
                i = t * G + e
                acc = sb[i, pl.ds(0, 16)] * db[i, pl.ds(0, 16)]
                for k in range(1, D // 16):
                    acc = acc + (sb[i, pl.ds(k * 16, 16)]
                                 * db[i, pl.ds(k * 16, 16)])
                fold[pl.ds(off + e * FR, G)] = acc
            stride = iota * FR + off
            tot = plsc.load_gather(fold, [stride])
            for k in range(1, G):
                tot = tot + plsc.load_gather(fold, [stride + k])
            outpair[pl.ds(q * C + t * G, G)] = 1.0 / (1.0 + jnp.exp(-tot))

    start(0, sidx0, didx0, srows0, drows0, sem0)

    def pair_body(p, _):
        j0 = 2 * p
        j1 = j0 + 1
        start(j1, sidx1, didx1, srows1, drows1, sem1)

        wait_rows(sidx0, didx0, srows0, drows0, sem0)
        compute(0, srows0, drows0, fold0)

        jn = lax.rem(j0 + 2, NCH)
        start(jn, sidx0, didx0, srows0, drows0, sem0)

        wait_rows(sidx1, didx1, srows1, drows1, sem1)
        compute(1, srows1, drows1, fold1)

        pltpu.sync_copy(outpair.at[pl.ds(0, 2 * C)],
                        out_hbm.at[pl.ds(base + p * 2 * C, 2 * C)])
        return 0

    lax.fori_loop(0, NP, pair_body, 0)
    wait_rows(sidx0, didx0, srows0, drows0, sem0)


@jax.jit
def kernel(z, edge_index):
    edge_index = edge_index.astype(jnp.int32)
    pad = jnp.zeros((2, CI - C), jnp.int32)
    edge_index = jnp.concatenate([edge_index, pad], axis=1)
    src = edge_index[0]
    dst = edge_index[1]

    mesh = plsc.VectorSubcoreMesh(core_axis_name="c", subcore_axis_name="s")
    run = functools.partial(
        pl.kernel,
        out_type=jax.ShapeDtypeStruct((E,), jnp.float32),
        mesh=mesh,
        compiler_params=pltpu.CompilerParams(needs_layout_passes=False),
        scratch_types=[
            pltpu.VMEM((CI,), jnp.int32),
            pltpu.VMEM((CI,), jnp.int32),
            pltpu.VMEM((CI,), jnp.int32),
            pltpu.VMEM((CI,), jnp.int32),
            pltpu.VMEM((CI, D), jnp.float32),
            pltpu.VMEM((CI, D), jnp.float32),
            pltpu.VMEM((CI, D), jnp.float32),
            pltpu.VMEM((CI, D), jnp.float32),
            pltpu.VMEM((2 * C + G,), jnp.float32),
            pltpu.VMEM((CI * FR,), jnp.float32),
            pltpu.VMEM((CI * FR,), jnp.float32),
            pltpu.SemaphoreType.DMA,
            pltpu.SemaphoreType.DMA,
        ],
    )(_sc_kernel)
    return run(z, src, dst)


# speedup vs baseline: 8.7169x; 1.1470x over previous
import functools

import jax
import jax.numpy as jnp
from jax import lax
from jax.experimental import pallas as pl
from jax.experimental.pallas import tpu as pltpu
from jax.experimental.pallas import tpu_sc as plsc

N_NODES = 10000
D = 128
E = 320000

NC = 2
NS = 16
NW = NC * NS
EPW = E // NW
C = 200
CI = 208
NCH = EPW // C
NP = NCH // 2
G = 16
FR = 17


def _sc_kernel(z_hbm, src_hbm, dst_hbm, out_hbm,
               sidx, didx,
               srows0, drows0, srows1, drows1,
               outpair, fold, sem0, sem1):
    wid = lax.axis_index("s") * NC + lax.axis_index("c")
    base = wid * EPW

    pltpu.sync_copy(src_hbm.at[pl.ds(base, EPW + 16)], sidx)
    pltpu.sync_copy(dst_hbm.at[pl.ds(base, EPW + 16)], didx)

    def start(j, sb, db, sem):
        pltpu.async_copy(z_hbm.at[sidx.at[pl.ds(j * C, CI)]], sb, sem)
        pltpu.async_copy(z_hbm.at[didx.at[pl.ds(j * C, CI)]], db, sem)

    def wait_rows(j, sb, db, sem):
        pltpu.make_async_copy(z_hbm.at[sidx.at[pl.ds(j * C, CI)]], sb, sem).wait()
        pltpu.make_async_copy(z_hbm.at[didx.at[pl.ds(j * C, CI)]], db, sem).wait()

    iota = lax.iota(jnp.int32, G)

    def compute(q, sb, db):
        @plsc.parallel_loop(0, CI // G, 1, unroll=1)
        def group_body(t):
            off = t * (G * FR)
            for e in range(G):
                i = t * G + e
                acc = sb[i, pl.ds(0, 16)] * db[i, pl.ds(0, 16)]
                for k in range(1, D // 16):
                    acc = acc + (sb[i, pl.ds(k * 16, 16)]
                                 * db[i, pl.ds(k * 16, 16)])
                fold[pl.ds(off + e * FR, G)] = acc
            stride = iota * FR + off
            tot = plsc.load_gather(fold, [stride])
            for k in range(1, G):
                tot = tot + plsc.load_gather(fold, [stride + k])
            outpair[pl.ds(q * C + t * G, G)] = 1.0 / (1.0 + jnp.exp(-tot))

    start(0, srows0, drows0, sem0)

    def pair_body(p, _):
        j0 = 2 * p
        j1 = j0 + 1
        start(j1, srows1, drows1, sem1)

        wait_rows(j0, srows0, drows0, sem0)
        compute(0, srows0, drows0)

        jn = lax.rem(j0 + 2, NCH)
        start(jn, srows0, drows0, sem0)

        wait_rows(j1, srows1, drows1, sem1)
        compute(1, srows1, drows1)

        pltpu.sync_copy(outpair.at[pl.ds(0, 2 * C)],
                        out_hbm.at[pl.ds(base + p * 2 * C, 2 * C)])
        return 0

    lax.fori_loop(0, NP, pair_body, 0)
    wait_rows(0, srows0, drows0, sem0)


@jax.jit
def kernel(z, edge_index):
    edge_index = edge_index.astype(jnp.int32)
    pad = jnp.zeros((2, 16), jnp.int32)
    edge_index = jnp.concatenate([edge_index, pad], axis=1)
    src = edge_index[0]
    dst = edge_index[1]

    mesh = plsc.VectorSubcoreMesh(core_axis_name="c", subcore_axis_name="s")
    run = functools.partial(
        pl.kernel,
        out_type=jax.ShapeDtypeStruct((E,), jnp.float32),
        mesh=mesh,
        compiler_params=pltpu.CompilerParams(needs_layout_passes=False),
        scratch_types=[
            pltpu.VMEM((EPW + 16,), jnp.int32),
            pltpu.VMEM((EPW + 16,), jnp.int32),
            pltpu.VMEM((CI, D), jnp.float32),
            pltpu.VMEM((CI, D), jnp.float32),
            pltpu.VMEM((CI, D), jnp.float32),
            pltpu.VMEM((CI, D), jnp.float32),
            pltpu.VMEM((2 * C + G,), jnp.float32),
            pltpu.VMEM((CI * FR,), jnp.float32),
            pltpu.SemaphoreType.DMA,
            pltpu.SemaphoreType.DMA,
        ],
    )(_sc_kernel)
    return run(z, src, dst)


# speedup vs baseline: 8.7469x; 1.0034x over previous
import functools

import jax
import jax.numpy as jnp
from jax import lax
from jax.experimental import pallas as pl
from jax.experimental.pallas import tpu as pltpu
from jax.experimental.pallas import tpu_sc as plsc

N_NODES = 10000
D = 128
E = 320000

NC = 2
NS = 16
NW = NC * NS
EPW = E // NW
C = 200
CI = 208
NCH = EPW // C
NP = NCH // 2
G = 16
FR = 17


def _sc_kernel(z_hbm, src_hbm, dst_hbm, out_hbm,
               sidx, didx,
               srows0, drows0, srows1, drows1,
               outpair, fold, sem0, sem1, semo):
    wid = lax.axis_index("s") * NC + lax.axis_index("c")
    base = wid * EPW

    pltpu.sync_copy(src_hbm.at[pl.ds(base, EPW + 16)], sidx)
    pltpu.sync_copy(dst_hbm.at[pl.ds(base, EPW + 16)], didx)

    def start(j, sb, db, sem):
        pltpu.async_copy(z_hbm.at[sidx.at[pl.ds(j * C, CI)]], sb, sem)
        pltpu.async_copy(z_hbm.at[didx.at[pl.ds(j * C, CI)]], db, sem)

    def wait_rows(j, sb, db, sem):
        pltpu.make_async_copy(z_hbm.at[sidx.at[pl.ds(j * C, CI)]], sb, sem).wait()
        pltpu.make_async_copy(z_hbm.at[didx.at[pl.ds(j * C, CI)]], db, sem).wait()

    iota = lax.iota(jnp.int32, G)

    def compute(q, sb, db):
        @plsc.parallel_loop(0, CI // G, 1, unroll=1)
        def group_body(t):
            off = t * (G * FR)
            for e in range(G):
                i = t * G + e
                acc = sb[i, pl.ds(0, 16)] * db[i, pl.ds(0, 16)]
                for k in range(1, D // 16):
                    acc = acc + (sb[i, pl.ds(k * 16, 16)]
                                 * db[i, pl.ds(k * 16, 16)])
                fold[pl.ds(off + e * FR, G)] = acc
            stride = iota * FR + off
            tot = plsc.load_gather(fold, [stride])
            for k in range(1, G):
                tot = tot + plsc.load_gather(fold, [stride + k])
            outpair[pl.ds(q * C + t * G, G)] = 1.0 / (1.0 + jnp.exp(-tot))

    def out_region(p):
        return out_hbm.at[pl.ds(base + p * 2 * C, 2 * C)]

    start(0, srows0, drows0, sem0)
    pltpu.async_copy(outpair.at[pl.ds(0, 2 * C)], out_region(0), semo)

    def pair_body(p, _):
        j0 = 2 * p
        j1 = j0 + 1
        start(j1, srows1, drows1, sem1)

        pltpu.make_async_copy(outpair.at[pl.ds(0, 2 * C)], out_region(0),
                              semo).wait()

        wait_rows(j0, srows0, drows0, sem0)
        compute(0, srows0, drows0)

        jn = lax.rem(j0 + 2, NCH)
        start(jn, srows0, drows0, sem0)

        wait_rows(j1, srows1, drows1, sem1)
        compute(1, srows1, drows1)

        pltpu.async_copy(outpair.at[pl.ds(0, 2 * C)], out_region(p), semo)
        return 0

    lax.fori_loop(0, NP, pair_body, 0)
    pltpu.make_async_copy(outpair.at[pl.ds(0, 2 * C)], out_region(0),
                          semo).wait()
    wait_rows(0, srows0, drows0, sem0)


@jax.jit
def kernel(z, edge_index):
    edge_index = edge_index.astype(jnp.int32)
    pad = jnp.zeros((2, 16), jnp.int32)
    edge_index = jnp.concatenate([edge_index, pad], axis=1)
    src = edge_index[0]
    dst = edge_index[1]

    mesh = plsc.VectorSubcoreMesh(core_axis_name="c", subcore_axis_name="s")
    run = functools.partial(
        pl.kernel,
        out_type=jax.ShapeDtypeStruct((E,), jnp.float32),
        mesh=mesh,
        compiler_params=pltpu.CompilerParams(needs_layout_passes=False),
        scratch_types=[
            pltpu.VMEM((EPW + 16,), jnp.int32),
            pltpu.VMEM((EPW + 16,), jnp.int32),
            pltpu.VMEM((CI, D), jnp.float32),
            pltpu.VMEM((CI, D), jnp.float32),
            pltpu.VMEM((CI, D), jnp.float32),
            pltpu.VMEM((CI, D), jnp.float32),
            pltpu.VMEM((2 * C + G,), jnp.float32),
            pltpu.VMEM((CI * FR,), jnp.float32),
            pltpu.SemaphoreType.DMA,
            pltpu.SemaphoreType.DMA,
            pltpu.SemaphoreType.DMA,
        ],
    )(_sc_kernel)
    return run(z, src, dst)


# speedup vs baseline: 8.8827x; 1.0155x over previous
import functools

import jax
import jax.numpy as jnp
from jax import lax
from jax.experimental import pallas as pl
from jax.experimental.pallas import tpu as pltpu
from jax.experimental.pallas import tpu_sc as plsc

N_NODES = 10000
D = 128
E = 320000

NC = 2
NS = 16
NW = NC * NS
EPW = E // NW
C = 200
CI = 208
NCH = EPW // C
NP = NCH // 2
G = 16
FR = 17


def _sc_kernel(z_hbm, src_hbm, dst_hbm, out_hbm,
               sidx, didx,
               srows0, drows0, srows1, drows1,
               outpair, fold, sem0, sem1, semo):
    wid = lax.axis_index("s") * NC + lax.axis_index("c")
    base = wid * EPW

    pltpu.sync_copy(src_hbm.at[pl.ds(base, EPW + 16)], sidx)
    pltpu.sync_copy(dst_hbm.at[pl.ds(base, EPW + 16)], didx)

    def start(j, sb, db, sem):
        pltpu.async_copy(z_hbm.at[sidx.at[pl.ds(j * C, C)]], sb.at[pl.ds(0, C)], sem)
        pltpu.async_copy(z_hbm.at[didx.at[pl.ds(j * C, C)]], db.at[pl.ds(0, C)], sem)

    def wait_rows(j, sb, db, sem):
        pltpu.make_async_copy(z_hbm.at[sidx.at[pl.ds(j * C, C)]], sb.at[pl.ds(0, C)], sem).wait()
        pltpu.make_async_copy(z_hbm.at[didx.at[pl.ds(j * C, C)]], db.at[pl.ds(0, C)], sem).wait()

    iota = lax.iota(jnp.int32, G)

    def compute(q, sb, db):
        @plsc.parallel_loop(0, CI // G, 1, unroll=1)
        def group_body(t):
            off = t * (G * FR)
            for e in range(G):
                i = t * G + e
                acc = sb[i, pl.ds(0, 16)] * db[i, pl.ds(0, 16)]
                for k in range(1, D // 16):
                    acc = acc + (sb[i, pl.ds(k * 16, 16)]
                                 * db[i, pl.ds(k * 16, 16)])
                fold[pl.ds(off + e * FR, G)] = acc
            stride = iota * FR + off
            tot = plsc.load_gather(fold, [stride])
            for k in range(1, G):
                tot = tot + plsc.load_gather(fold, [stride + k])
            outpair[pl.ds(q * C + t * G, G)] = 1.0 / (1.0 + jnp.exp(-tot))

    def out_region(p):
        return out_hbm.at[pl.ds(base + p * 2 * C, 2 * C)]

    start(0, srows0, drows0, sem0)
    pltpu.async_copy(outpair.at[pl.ds(0, 2 * C)], out_region(0), semo)

    def pair_body(p, _):
        j0 = 2 * p
        j1 = j0 + 1
        start(j1, srows1, drows1, sem1)

        pltpu.make_async_copy(outpair.at[pl.ds(0, 2 * C)], out_region(0),
                              semo).wait()

        wait_rows(j0, srows0, drows0, sem0)
        compute(0, srows0, drows0)

        jn = lax.rem(j0 + 2, NCH)
        start(jn, srows0, drows0, sem0)

        wait_rows(j1, srows1, drows1, sem1)
        compute(1, srows1, drows1)

        pltpu.async_copy(outpair.at[pl.ds(0, 2 * C)], out_region(p), semo)
        return 0

    lax.fori_loop(0, NP, pair_body, 0)
    pltpu.make_async_copy(outpair.at[pl.ds(0, 2 * C)], out_region(0),
                          semo).wait()
    wait_rows(0, srows0, drows0, sem0)


@jax.jit
def kernel(z, edge_index):
    edge_index = edge_index.astype(jnp.int32)
    pad = jnp.zeros((2, 16), jnp.int32)
    edge_index = jnp.concatenate([edge_index, pad], axis=1)
    src = edge_index[0]
    dst = edge_index[1]

    mesh = plsc.VectorSubcoreMesh(core_axis_name="c", subcore_axis_name="s")
    run = functools.partial(
        pl.kernel,
        out_type=jax.ShapeDtypeStruct((E,), jnp.float32),
        mesh=mesh,
        compiler_params=pltpu.CompilerParams(needs_layout_passes=False),
        scratch_types=[
            pltpu.VMEM((EPW + 16,), jnp.int32),
            pltpu.VMEM((EPW + 16,), jnp.int32),
            pltpu.VMEM((CI, D), jnp.float32),
            pltpu.VMEM((CI, D), jnp.float32),
            pltpu.VMEM((CI, D), jnp.float32),
            pltpu.VMEM((CI, D), jnp.float32),
            pltpu.VMEM((2 * C + G,), jnp.float32),
            pltpu.VMEM((CI * FR,), jnp.float32),
            pltpu.SemaphoreType.DMA,
            pltpu.SemaphoreType.DMA,
            pltpu.SemaphoreType.DMA,
        ],
    )(_sc_kernel)
    return run(z, src, dst)
